# baseline (speedup 1.0000x reference)
import functools
import jax
import jax.numpy as jnp
from jax import lax
from jax.experimental import pallas as pl
from jax.experimental.pallas import tpu as pltpu
from jax.experimental.pallas import tpu_sc as plsc

N = 10000
E = 320000
F_IN = 128
HID = 32
T = 4

BLK = 2000
NB = N // BLK

NW = 32
EPT = E // NW
EB = 200
NBLK = EPT // EB
RB = 624
RT = N - 16 * RB
ZC = 48


def _sc_mesh():
    return plsc.VectorSubcoreMesh(core_axis_name="c", subcore_axis_name="s")


def _deg_kernel(srcr):
    @functools.partial(
        pl.kernel,
        mesh=_sc_mesh(),
        out_type=jax.ShapeDtypeStruct((2, N, 16), jnp.float32),
        scratch_types=[
            pltpu.VMEM((NBLK, EB), jnp.int32),
            pltpu.VMEM((EB, 16), jnp.float32),
            pltpu.VMEM((ZC, 16), jnp.float32),
            pltpu.VMEM_SHARED((N, 16), jnp.float32),
        ],
        compiler_params=pltpu.CompilerParams(use_tc_tiling_on_sc=False),
    )
    def k(srcr_hbm, out_hbm, src_v, ones_v, zrow_v, acc):
        c = lax.axis_index("c")
        s = lax.axis_index("s")
        wid = c * 16 + s
        pltpu.sync_copy(srcr_hbm.at[wid], src_v)

        def fill(i, carry):
            ones_v[i, :] = jnp.ones((16,), jnp.float32)
            return carry
        lax.fori_loop(0, EB, fill, 0)

        def zfill(i, carry):
            zrow_v[i, :] = jnp.zeros((16,), jnp.float32)
            return carry
        lax.fori_loop(0, ZC, zfill, 0)
        for kk in range(RB // ZC):
            pltpu.sync_copy(zrow_v, acc.at[pl.ds(s * RB + kk * ZC, ZC)])

        @pl.when(s == 15)
        def _():
            pltpu.sync_copy(zrow_v.at[pl.ds(0, RT)], acc.at[pl.ds(16 * RB, RT)])
        plsc.subcore_barrier()

        def body(j, carry):
            pltpu.sync_copy(ones_v, acc.at[src_v.at[j]], add=True)
            return carry
        lax.fori_loop(0, NBLK, body, 0)
        plsc.subcore_barrier()
        pltpu.sync_copy(acc.at[pl.ds(s * RB, RB)],
                        out_hbm.at[c, pl.ds(s * RB, RB)])

        @pl.when(s == 15)
        def _():
            pltpu.sync_copy(acc.at[pl.ds(16 * RB, RT)],
                            out_hbm.at[c, pl.ds(16 * RB, RT)])

    return k(srcr)


def _make_sc_pass(W, ntab, nbuf):
    @functools.partial(
        pl.kernel,
        mesh=_sc_mesh(),
        out_type=[jax.ShapeDtypeStruct((2, N, W), jnp.float32)] * ntab,
        scratch_types=[
            pltpu.VMEM((NBLK, EB), jnp.int32),
            pltpu.VMEM((NBLK, EB), jnp.int32),
            pltpu.VMEM((nbuf, EB, W), jnp.float32),
            pltpu.VMEM((ZC, W), jnp.float32),
            pltpu.VMEM_SHARED((N, W), jnp.float32),
        ] + [pltpu.SemaphoreType.DMA] * nbuf,
        compiler_params=pltpu.CompilerParams(use_tc_tiling_on_sc=False),
    )
    def k(*args):
        tabs = args[:ntab]
        srcr, dstr = args[ntab], args[ntab + 1]
        outs = args[ntab + 2:2 * ntab + 2]
        src_v, dst_v, bufs, zrow_v, acc = args[2 * ntab + 2:2 * ntab + 7]
        sems = args[2 * ntab + 7:]
        c = lax.axis_index("c")
        s = lax.axis_index("s")
        wid = c * 16 + s
        pltpu.sync_copy(srcr.at[wid], src_v)
        pltpu.sync_copy(dstr.at[wid], dst_v)

        def zfill(i, carry):
            for kk in range(W // 16):
                zrow_v[i, pl.ds(kk * 16, 16)] = jnp.zeros((16,), jnp.float32)
            return carry
        lax.fori_loop(0, ZC, zfill, 0)

        for ti in range(ntab):
            tab = tabs[ti]
            for b in range(nbuf):
                pltpu.async_copy(tab.at[src_v.at[b]], bufs.at[b], sems[b])

            for kk in range(RB // ZC):
                pltpu.sync_copy(zrow_v, acc.at[pl.ds(s * RB + kk * ZC, ZC)])

            @pl.when(s == 15)
            def _():
                pltpu.sync_copy(zrow_v.at[pl.ds(0, RT)],
                                acc.at[pl.ds(16 * RB, RT)])
            plsc.subcore_barrier()

            def body(g, carry):
                for b in range(nbuf):
                    j = g * nbuf + b
                    pltpu.make_async_copy(
                        tab.at[pl.ds(0, EB)], bufs.at[b], sems[b]).wait()
                    pltpu.sync_copy(bufs.at[b], acc.at[dst_v.at[j]], add=True)

                    @pl.when(g < NBLK // nbuf - 1)
                    def _(b=b, j=j):
                        pltpu.async_copy(tab.at[src_v.at[j + nbuf]],
                                         bufs.at[b], sems[b])
                return carry
            lax.fori_loop(0, NBLK // nbuf, body, 0)
            plsc.subcore_barrier()
            pltpu.sync_copy(acc.at[pl.ds(s * RB, RB)],
                            outs[ti].at[c, pl.ds(s * RB, RB)])

            @pl.when(s == 15)
            def _():
                pltpu.sync_copy(acc.at[pl.ds(16 * RB, RT)],
                                outs[ti].at[c, pl.ds(16 * RB, RT)])
            plsc.subcore_barrier()

    return k


_sc_pass4 = _make_sc_pass(3 * HID, 4, 2)
_sc_pass1 = _make_sc_pass(HID, 1, 5)


def _dinv_of(degp_ref):
    deg = degp_ref[0, :, 0:1] + degp_ref[1, :, 0:1]
    return jnp.where(deg > 0, lax.rsqrt(deg), 0.0)


def _prep_body(x_ref, w0_ref, w1_ref, b_ref, degp_ref, xp0_ref, xpre_ref):
    xb = x_ref[0]
    dinv = _dinv_of(degp_ref)
    xp0_ref[0] = jnp.dot(xb, w0_ref[...], preferred_element_type=jnp.float32) + b_ref[...]
    xpre_ref[0] = dinv * jnp.dot(xb, w1_ref[...], preferred_element_type=jnp.float32)


def _prep(x, w0cat, w1cat, bcat, degp):
    return pl.pallas_call(
        _prep_body,
        grid=(T, NB),
        in_specs=[
            pl.BlockSpec((1, BLK, F_IN), lambda t, i: (t, i, 0)),
            pl.BlockSpec((F_IN, 3 * HID), lambda t, i: (0, 0)),
            pl.BlockSpec((F_IN, 3 * HID), lambda t, i: (0, 0)),
            pl.BlockSpec((1, 3 * HID), lambda t, i: (0, 0)),
            pl.BlockSpec((2, BLK, 16), lambda t, i: (0, i, 0)),
        ],
        out_specs=[
            pl.BlockSpec((1, BLK, 3 * HID), lambda t, i: (t, i, 0)),
            pl.BlockSpec((1, BLK, 3 * HID), lambda t, i: (t, i, 0)),
        ],
        out_shape=[
            jax.ShapeDtypeStruct((T, N, 3 * HID), jnp.float32),
            jax.ShapeDtypeStruct((T, N, 3 * HID), jnp.float32),
        ],
    )(x, w0cat, w1cat, bcat, degp)


def _t0_body(xp0_ref, sx_ref, degp_ref, h_ref, hp_ref):
    dinv = _dinv_of(degp_ref)
    pre = xp0_ref[0] - dinv * (sx_ref[0] + sx_ref[1])
    z = jax.nn.sigmoid(pre[:, 0:HID])
    ht = jnp.tanh(pre[:, 2 * HID:3 * HID])
    h = (1.0 - z) * ht
    h_ref[...] = h
    hp_ref[...] = dinv * h


def _t0(xp0, sx, degp):
    return pl.pallas_call(
        _t0_body,
        grid=(NB,),
        in_specs=[
            pl.BlockSpec((1, BLK, 3 * HID), lambda i: (0, i, 0)),
            pl.BlockSpec((2, BLK, 3 * HID), lambda i: (0, i, 0)),
            pl.BlockSpec((2, BLK, 16), lambda i: (0, i, 0)),
        ],
        out_specs=[
            pl.BlockSpec((BLK, HID), lambda i: (i, 0)),
            pl.BlockSpec((BLK, HID), lambda i: (i, 0)),
        ],
        out_shape=[
            jax.ShapeDtypeStruct((N, HID), jnp.float32),
            jax.ShapeDtypeStruct((N, HID), jnp.float32),
        ],
    )(xp0, sx, degp)


def _gates_body(xp0_ref, sx_ref, h_ref, sh_ref, degp_ref,
                wz0_ref, wr0_ref, wz1_ref, wr1_ref,
                z_ref, g_ref, gp_ref):
    dinv = _dinv_of(degp_ref)
    h = h_ref[...]
    lh = -dinv * (sh_ref[0] + sh_ref[1])
    xp = xp0_ref[0] - dinv * (sx_ref[0] + sx_ref[1])
    z = jax.nn.sigmoid(
        xp[:, 0:HID]
        + jnp.dot(h, wz0_ref[...], preferred_element_type=jnp.float32)
        + jnp.dot(lh, wz1_ref[...], preferred_element_type=jnp.float32))
    r = jax.nn.sigmoid(
        xp[:, HID:2 * HID]
        + jnp.dot(h, wr0_ref[...], preferred_element_type=jnp.float32)
        + jnp.dot(lh, wr1_ref[...], preferred_element_type=jnp.float32))
    g = h * r
    z_ref[...] = z
    g_ref[...] = g
    gp_ref[...] = dinv * g


def _gates(t, xp0, sx, h, sh, degp, wz0, wr0, wz1, wr1):
    return pl.pallas_call(
        _gates_body,
        grid=(NB,),
        in_specs=[
            pl.BlockSpec((1, BLK, 3 * HID), lambda i, t=t: (t, i, 0)),
            pl.BlockSpec((2, BLK, 3 * HID), lambda i: (0, i, 0)),
            pl.BlockSpec((BLK, HID), lambda i: (i, 0)),
            pl.BlockSpec((2, BLK, HID), lambda i: (0, i, 0)),
            pl.BlockSpec((2, BLK, 16), lambda i: (0, i, 0)),
            pl.BlockSpec((HID, HID), lambda i: (0, 0)),
            pl.BlockSpec((HID, HID), lambda i: (0, 0)),
            pl.BlockSpec((HID, HID), lambda i: (0, 0)),
            pl.BlockSpec((HID, HID), lambda i: (0, 0)),
        ],
        out_specs=[
            pl.BlockSpec((BLK, HID), lambda i: (i, 0)),
            pl.BlockSpec((BLK, HID), lambda i: (i, 0)),
            pl.BlockSpec((BLK, HID), lambda i: (i, 0)),
        ],
        out_shape=[
            jax.ShapeDtypeStruct((N, HID), jnp.float32),
            jax.ShapeDtypeStruct((N, HID), jnp.float32),
            jax.ShapeDtypeStruct((N, HID), jnp.float32),
        ],
    )(xp0, sx, h, sh, degp, wz0, wr0, wz1, wr1)


def _update_body(xp0_ref, sx_ref, z_ref, g_ref, sg_ref, h_ref, degp_ref,
                 wh0_ref, wh1_ref, hn_ref, hp_ref):
    dinv = _dinv_of(degp_ref)
    g = g_ref[...]
    lg = -dinv * (sg_ref[0] + sg_ref[1])
    xh = (xp0_ref[0, :, 2 * HID:3 * HID]
          - dinv * (sx_ref[0, :, 2 * HID:3 * HID] + sx_ref[1, :, 2 * HID:3 * HID]))
    ht = jnp.tanh(
        xh
        + jnp.dot(g, wh0_ref[...], preferred_element_type=jnp.float32)
        + jnp.dot(lg, wh1_ref[...], preferred_element_type=jnp.float32))
    z = z_ref[...]
    hn = z * h_ref[...] + (1.0 - z) * ht
    hn_ref[...] = hn
    hp_ref[...] = dinv * hn


def _update(t, xp0, sx, z, g, sg, h, degp, wh0, wh1):
    return pl.pallas_call(
        _update_body,
        grid=(NB,),
        in_specs=[
            pl.BlockSpec((1, BLK, 3 * HID), lambda i, t=t: (t, i, 0)),
            pl.BlockSpec((2, BLK, 3 * HID), lambda i: (0, i, 0)),
            pl.BlockSpec((BLK, HID), lambda i: (i, 0)),
            pl.BlockSpec((BLK, HID), lambda i: (i, 0)),
            pl.BlockSpec((2, BLK, HID), lambda i: (0, i, 0)),
            pl.BlockSpec((BLK, HID), lambda i: (i, 0)),
            pl.BlockSpec((2, BLK, 16), lambda i: (0, i, 0)),
            pl.BlockSpec((HID, HID), lambda i: (0, 0)),
            pl.BlockSpec((HID, HID), lambda i: (0, 0)),
        ],
        out_specs=[
            pl.BlockSpec((BLK, HID), lambda i: (i, 0)),
            pl.BlockSpec((BLK, HID), lambda i: (i, 0)),
        ],
        out_shape=[
            jax.ShapeDtypeStruct((N, HID), jnp.float32),
            jax.ShapeDtypeStruct((N, HID), jnp.float32),
        ],
    )(xp0, sx, z, g, sg, h, degp, wh0, wh1)


def _dec_body(h_ref, w1_ref, b1_ref, w2_ref, b2_ref, out_ref):
    h1 = jax.nn.relu(
        jnp.dot(h_ref[...], w1_ref[...], preferred_element_type=jnp.float32)
        + b1_ref[...])
    out_ref[...] = jnp.dot(h1, w2_ref[...], preferred_element_type=jnp.float32) + b2_ref[...]


def _decoder(h, w1, b1, w2, b2):
    return pl.pallas_call(
        _dec_body,
        grid=(NB,),
        in_specs=[
            pl.BlockSpec((BLK, HID), lambda i: (i, 0)),
            pl.BlockSpec((HID, HID), lambda i: (0, 0)),
            pl.BlockSpec((1, HID), lambda i: (0, 0)),
            pl.BlockSpec((HID, F_IN), lambda i: (0, 0)),
            pl.BlockSpec((1, F_IN), lambda i: (0, 0)),
        ],
        out_specs=pl.BlockSpec((BLK, F_IN), lambda i: (i, 0)),
        out_shape=jax.ShapeDtypeStruct((N, F_IN), jnp.float32),
    )(h, w1, b1, w2, b2)


def kernel(x, edge_index, params):
    srcr = edge_index[0].reshape(NW, NBLK, EB)
    dstr = edge_index[1].reshape(NW, NBLK, EB)

    degp = _deg_kernel(srcr)

    w0cat = jnp.concatenate(
        [params['x_z']['W'][0], params['x_r']['W'][0], params['x_h']['W'][0]], axis=1)
    w1cat = jnp.concatenate(
        [params['x_z']['W'][1], params['x_r']['W'][1], params['x_h']['W'][1]], axis=1)
    bcat = jnp.concatenate(
        [params['x_z']['b'] + params['h_z']['b'],
         params['x_r']['b'] + params['h_r']['b'],
         params['x_h']['b'] + params['h_h']['b']]).reshape(1, 3 * HID)

    xp0, xpre = _prep(x, w0cat, w1cat, bcat, degp)

    sxp = _sc_pass4(xpre[0], xpre[1], xpre[2], xpre[3], srcr, dstr)

    h, hp = _t0(xp0, sxp[0], degp)

    for t in range(1, T):
        (shp,) = _sc_pass1(hp, srcr, dstr)
        z, g, gp = _gates(t, xp0, sxp[t], h, shp, degp,
                          params['h_z']['W'][0], params['h_r']['W'][0],
                          params['h_z']['W'][1], params['h_r']['W'][1])
        (sgp,) = _sc_pass1(gp, srcr, dstr)
        h, hp = _update(t, xp0, sxp[t], z, g, sgp, h, degp,
                        params['h_h']['W'][0], params['h_h']['W'][1])

    x_pred = _decoder(h, params['dec_W1'], params['dec_b1'].reshape(1, HID),
                      params['dec_W2'], params['dec_b2'].reshape(1, F_IN))
    return (x_pred, h)


# speedup vs baseline: 1.0099x; 1.0099x over previous
import functools
import jax
import jax.numpy as jnp
from jax import lax
from jax.experimental import pallas as pl
from jax.experimental.pallas import tpu as pltpu
from jax.experimental.pallas import tpu_sc as plsc

N = 10000
E = 320000
F_IN = 128
HID = 32
T = 4

BLK = 2000
NB = N // BLK

NW = 32
EPT = E // NW
EB = 200
NBLK = EPT // EB
RB = 624
RT = N - 16 * RB
ZC = 48


def _sc_mesh():
    return plsc.VectorSubcoreMesh(core_axis_name="c", subcore_axis_name="s")


def _deg_kernel(srcr):
    @functools.partial(
        pl.kernel,
        mesh=_sc_mesh(),
        out_type=jax.ShapeDtypeStruct((2, N, 16), jnp.float32),
        scratch_types=[
            pltpu.VMEM((NBLK, EB), jnp.int32),
            pltpu.VMEM((EB, 16), jnp.float32),
            pltpu.VMEM((ZC, 16), jnp.float32),
            pltpu.VMEM_SHARED((N, 16), jnp.float32),
        ],
        compiler_params=pltpu.CompilerParams(use_tc_tiling_on_sc=False),
    )
    def k(srcr_hbm, out_hbm, src_v, ones_v, zrow_v, acc):
        c = lax.axis_index("c")
        s = lax.axis_index("s")
        wid = c * 16 + s
        pltpu.sync_copy(srcr_hbm.at[wid], src_v)

        def fill(i, carry):
            ones_v[i, :] = jnp.ones((16,), jnp.float32)
            return carry
        lax.fori_loop(0, EB, fill, 0)

        def zfill(i, carry):
            zrow_v[i, :] = jnp.zeros((16,), jnp.float32)
            return carry
        lax.fori_loop(0, ZC, zfill, 0)
        for kk in range(RB // ZC):
            pltpu.sync_copy(zrow_v, acc.at[pl.ds(s * RB + kk * ZC, ZC)])

        @pl.when(s == 15)
        def _():
            pltpu.sync_copy(zrow_v.at[pl.ds(0, RT)], acc.at[pl.ds(16 * RB, RT)])
        plsc.subcore_barrier()

        def body(j, carry):
            pltpu.sync_copy(ones_v, acc.at[src_v.at[j]], add=True)
            return carry
        lax.fori_loop(0, NBLK, body, 0)
        plsc.subcore_barrier()
        pltpu.sync_copy(acc.at[pl.ds(s * RB, RB)],
                        out_hbm.at[c, pl.ds(s * RB, RB)])

        @pl.when(s == 15)
        def _():
            pltpu.sync_copy(acc.at[pl.ds(16 * RB, RT)],
                            out_hbm.at[c, pl.ds(16 * RB, RT)])

    return k(srcr)


def _make_sc_pass(W, ntab, nbuf):
    @functools.partial(
        pl.kernel,
        mesh=_sc_mesh(),
        out_type=[jax.ShapeDtypeStruct((2, N, W), jnp.float32)] * ntab,
        scratch_types=[
            pltpu.VMEM((NBLK, EB), jnp.int32),
            pltpu.VMEM((NBLK, EB), jnp.int32),
            pltpu.VMEM((nbuf, EB, W), jnp.float32),
            pltpu.VMEM((ZC, W), jnp.float32),
            pltpu.VMEM_SHARED((N, W), jnp.float32),
        ] + [pltpu.SemaphoreType.DMA] * nbuf,
        compiler_params=pltpu.CompilerParams(use_tc_tiling_on_sc=False),
    )
    def k(*args):
        tabs = args[:ntab]
        srcr, dstr = args[ntab], args[ntab + 1]
        outs = args[ntab + 2:2 * ntab + 2]
        src_v, dst_v, bufs, zrow_v, acc = args[2 * ntab + 2:2 * ntab + 7]
        sems = args[2 * ntab + 7:]
        c = lax.axis_index("c")
        s = lax.axis_index("s")
        wid = c * 16 + s
        pltpu.sync_copy(srcr.at[wid], src_v)
        pltpu.sync_copy(dstr.at[wid], dst_v)

        def zfill(i, carry):
            for kk in range(W // 16):
                zrow_v[i, pl.ds(kk * 16, 16)] = jnp.zeros((16,), jnp.float32)
            return carry
        lax.fori_loop(0, ZC, zfill, 0)

        for ti in range(ntab):
            tab = tabs[ti]
            for b in range(nbuf):
                pltpu.async_copy(tab.at[src_v.at[b]], bufs.at[b], sems[b])

            for kk in range(RB // ZC):
                pltpu.sync_copy(zrow_v, acc.at[pl.ds(s * RB + kk * ZC, ZC)])

            @pl.when(s == 15)
            def _():
                pltpu.sync_copy(zrow_v.at[pl.ds(0, RT)],
                                acc.at[pl.ds(16 * RB, RT)])
            plsc.subcore_barrier()

            def body(g, carry):
                for b in range(nbuf):
                    j = g * nbuf + b
                    pltpu.make_async_copy(
                        tab.at[pl.ds(0, EB)], bufs.at[b], sems[b]).wait()
                    pltpu.sync_copy(bufs.at[b], acc.at[dst_v.at[j]], add=True)

                    @pl.when(g < NBLK // nbuf - 1)
                    def _(b=b, j=j):
                        pltpu.async_copy(tab.at[src_v.at[j + nbuf]],
                                         bufs.at[b], sems[b])
                return carry
            lax.fori_loop(0, NBLK // nbuf, body, 0)
            plsc.subcore_barrier()
            pltpu.sync_copy(acc.at[pl.ds(s * RB, RB)],
                            outs[ti].at[c, pl.ds(s * RB, RB)])

            @pl.when(s == 15)
            def _():
                pltpu.sync_copy(acc.at[pl.ds(16 * RB, RT)],
                                outs[ti].at[c, pl.ds(16 * RB, RT)])
            plsc.subcore_barrier()

    return k


_sc_pass4 = _make_sc_pass(3 * HID, 4, 2)
_sc_pass1 = _make_sc_pass(HID, 1, 5)


def _dinv_of(degp_ref):
    deg = degp_ref[0, :, 0:1] + degp_ref[1, :, 0:1]
    return jnp.where(deg > 0, lax.rsqrt(deg), 0.0)


def _prep_body(x_ref, w0_ref, w1_ref, b_ref, degp_ref, xp0_ref, xpre_ref):
    xb = x_ref[0]
    dinv = _dinv_of(degp_ref)
    xp0_ref[0] = jnp.dot(xb, w0_ref[...], preferred_element_type=jnp.float32) + b_ref[...]
    xpre_ref[0] = dinv * jnp.dot(xb, w1_ref[...], preferred_element_type=jnp.float32)


def _prep(x, w0cat, w1cat, bcat, degp):
    return pl.pallas_call(
        _prep_body,
        grid=(T, NB),
        in_specs=[
            pl.BlockSpec((1, BLK, F_IN), lambda t, i: (t, i, 0)),
            pl.BlockSpec((F_IN, 3 * HID), lambda t, i: (0, 0)),
            pl.BlockSpec((F_IN, 3 * HID), lambda t, i: (0, 0)),
            pl.BlockSpec((1, 3 * HID), lambda t, i: (0, 0)),
            pl.BlockSpec((2, BLK, 16), lambda t, i: (0, i, 0)),
        ],
        out_specs=[
            pl.BlockSpec((1, BLK, 3 * HID), lambda t, i: (t, i, 0)),
            pl.BlockSpec((1, BLK, 3 * HID), lambda t, i: (t, i, 0)),
        ],
        out_shape=[
            jax.ShapeDtypeStruct((T, N, 3 * HID), jnp.float32),
            jax.ShapeDtypeStruct((T, N, 3 * HID), jnp.float32),
        ],
    )(x, w0cat, w1cat, bcat, degp)


def _t0_body(xp0_ref, sx_ref, degp_ref, h_ref, hp_ref):
    dinv = _dinv_of(degp_ref)
    pre = xp0_ref[0] - dinv * (sx_ref[0] + sx_ref[1])
    z = jax.nn.sigmoid(pre[:, 0:HID])
    ht = jnp.tanh(pre[:, 2 * HID:3 * HID])
    h = (1.0 - z) * ht
    h_ref[...] = h
    hp_ref[...] = dinv * h


def _t0(xp0, sx, degp):
    return pl.pallas_call(
        _t0_body,
        grid=(NB,),
        in_specs=[
            pl.BlockSpec((1, BLK, 3 * HID), lambda i: (0, i, 0)),
            pl.BlockSpec((2, BLK, 3 * HID), lambda i: (0, i, 0)),
            pl.BlockSpec((2, BLK, 16), lambda i: (0, i, 0)),
        ],
        out_specs=[
            pl.BlockSpec((BLK, HID), lambda i: (i, 0)),
            pl.BlockSpec((BLK, HID), lambda i: (i, 0)),
        ],
        out_shape=[
            jax.ShapeDtypeStruct((N, HID), jnp.float32),
            jax.ShapeDtypeStruct((N, HID), jnp.float32),
        ],
    )(xp0, sx, degp)


def _gates_body(xp0_ref, sx_ref, h_ref, sh_ref, degp_ref,
                wz0_ref, wr0_ref, wz1_ref, wr1_ref,
                z_ref, g_ref, gp_ref):
    dinv = _dinv_of(degp_ref)
    h = h_ref[...]
    lh = -dinv * (sh_ref[0] + sh_ref[1])
    xp = xp0_ref[0] - dinv * (sx_ref[0] + sx_ref[1])
    z = jax.nn.sigmoid(
        xp[:, 0:HID]
        + jnp.dot(h, wz0_ref[...], preferred_element_type=jnp.float32)
        + jnp.dot(lh, wz1_ref[...], preferred_element_type=jnp.float32))
    r = jax.nn.sigmoid(
        xp[:, HID:2 * HID]
        + jnp.dot(h, wr0_ref[...], preferred_element_type=jnp.float32)
        + jnp.dot(lh, wr1_ref[...], preferred_element_type=jnp.float32))
    g = h * r
    z_ref[...] = z
    g_ref[...] = g
    gp_ref[...] = dinv * g


def _gates(t, xp0, sx, h, sh, degp, wz0, wr0, wz1, wr1):
    return pl.pallas_call(
        _gates_body,
        grid=(NB,),
        in_specs=[
            pl.BlockSpec((1, BLK, 3 * HID), lambda i, t=t: (t, i, 0)),
            pl.BlockSpec((2, BLK, 3 * HID), lambda i: (0, i, 0)),
            pl.BlockSpec((BLK, HID), lambda i: (i, 0)),
            pl.BlockSpec((2, BLK, HID), lambda i: (0, i, 0)),
            pl.BlockSpec((2, BLK, 16), lambda i: (0, i, 0)),
            pl.BlockSpec((HID, HID), lambda i: (0, 0)),
            pl.BlockSpec((HID, HID), lambda i: (0, 0)),
            pl.BlockSpec((HID, HID), lambda i: (0, 0)),
            pl.BlockSpec((HID, HID), lambda i: (0, 0)),
        ],
        out_specs=[
            pl.BlockSpec((BLK, HID), lambda i: (i, 0)),
            pl.BlockSpec((BLK, HID), lambda i: (i, 0)),
            pl.BlockSpec((BLK, HID), lambda i: (i, 0)),
        ],
        out_shape=[
            jax.ShapeDtypeStruct((N, HID), jnp.float32),
            jax.ShapeDtypeStruct((N, HID), jnp.float32),
            jax.ShapeDtypeStruct((N, HID), jnp.float32),
        ],
    )(xp0, sx, h, sh, degp, wz0, wr0, wz1, wr1)


def _update_body(xp0_ref, sx_ref, z_ref, g_ref, sg_ref, h_ref, degp_ref,
                 wh0_ref, wh1_ref, hn_ref, hp_ref):
    dinv = _dinv_of(degp_ref)
    g = g_ref[...]
    lg = -dinv * (sg_ref[0] + sg_ref[1])
    xh = (xp0_ref[0, :, 2 * HID:3 * HID]
          - dinv * (sx_ref[0, :, 2 * HID:3 * HID] + sx_ref[1, :, 2 * HID:3 * HID]))
    ht = jnp.tanh(
        xh
        + jnp.dot(g, wh0_ref[...], preferred_element_type=jnp.float32)
        + jnp.dot(lg, wh1_ref[...], preferred_element_type=jnp.float32))
    z = z_ref[...]
    hn = z * h_ref[...] + (1.0 - z) * ht
    hn_ref[...] = hn
    hp_ref[...] = dinv * hn


def _update(t, xp0, sx, z, g, sg, h, degp, wh0, wh1):
    return pl.pallas_call(
        _update_body,
        grid=(NB,),
        in_specs=[
            pl.BlockSpec((1, BLK, 3 * HID), lambda i, t=t: (t, i, 0)),
            pl.BlockSpec((2, BLK, 3 * HID), lambda i: (0, i, 0)),
            pl.BlockSpec((BLK, HID), lambda i: (i, 0)),
            pl.BlockSpec((BLK, HID), lambda i: (i, 0)),
            pl.BlockSpec((2, BLK, HID), lambda i: (0, i, 0)),
            pl.BlockSpec((BLK, HID), lambda i: (i, 0)),
            pl.BlockSpec((2, BLK, 16), lambda i: (0, i, 0)),
            pl.BlockSpec((HID, HID), lambda i: (0, 0)),
            pl.BlockSpec((HID, HID), lambda i: (0, 0)),
        ],
        out_specs=[
            pl.BlockSpec((BLK, HID), lambda i: (i, 0)),
            pl.BlockSpec((BLK, HID), lambda i: (i, 0)),
        ],
        out_shape=[
            jax.ShapeDtypeStruct((N, HID), jnp.float32),
            jax.ShapeDtypeStruct((N, HID), jnp.float32),
        ],
    )(xp0, sx, z, g, sg, h, degp, wh0, wh1)


def _final_body(xp0_ref, sx_ref, z_ref, g_ref, sg_ref, h_ref, degp_ref,
                wh0_ref, wh1_ref, w1_ref, b1_ref, w2_ref, b2_ref,
                hn_ref, out_ref):
    dinv = _dinv_of(degp_ref)
    g = g_ref[...]
    lg = -dinv * (sg_ref[0] + sg_ref[1])
    xh = (xp0_ref[0, :, 2 * HID:3 * HID]
          - dinv * (sx_ref[0, :, 2 * HID:3 * HID] + sx_ref[1, :, 2 * HID:3 * HID]))
    ht = jnp.tanh(
        xh
        + jnp.dot(g, wh0_ref[...], preferred_element_type=jnp.float32)
        + jnp.dot(lg, wh1_ref[...], preferred_element_type=jnp.float32))
    z = z_ref[...]
    hn = z * h_ref[...] + (1.0 - z) * ht
    hn_ref[...] = hn
    h1 = jax.nn.relu(
        jnp.dot(hn, w1_ref[...], preferred_element_type=jnp.float32)
        + b1_ref[...])
    out_ref[...] = jnp.dot(h1, w2_ref[...], preferred_element_type=jnp.float32) + b2_ref[...]


def _final(t, xp0, sx, z, g, sg, h, degp, wh0, wh1, w1, b1, w2, b2):
    return pl.pallas_call(
        _final_body,
        grid=(NB,),
        in_specs=[
            pl.BlockSpec((1, BLK, 3 * HID), lambda i, t=t: (t, i, 0)),
            pl.BlockSpec((2, BLK, 3 * HID), lambda i: (0, i, 0)),
            pl.BlockSpec((BLK, HID), lambda i: (i, 0)),
            pl.BlockSpec((BLK, HID), lambda i: (i, 0)),
            pl.BlockSpec((2, BLK, HID), lambda i: (0, i, 0)),
            pl.BlockSpec((BLK, HID), lambda i: (i, 0)),
            pl.BlockSpec((2, BLK, 16), lambda i: (0, i, 0)),
            pl.BlockSpec((HID, HID), lambda i: (0, 0)),
            pl.BlockSpec((HID, HID), lambda i: (0, 0)),
            pl.BlockSpec((HID, HID), lambda i: (0, 0)),
            pl.BlockSpec((1, HID), lambda i: (0, 0)),
            pl.BlockSpec((HID, F_IN), lambda i: (0, 0)),
            pl.BlockSpec((1, F_IN), lambda i: (0, 0)),
        ],
        out_specs=[
            pl.BlockSpec((BLK, HID), lambda i: (i, 0)),
            pl.BlockSpec((BLK, F_IN), lambda i: (i, 0)),
        ],
        out_shape=[
            jax.ShapeDtypeStruct((N, HID), jnp.float32),
            jax.ShapeDtypeStruct((N, F_IN), jnp.float32),
        ],
    )(xp0, sx, z, g, sg, h, degp, wh0, wh1, w1, b1, w2, b2)


def kernel(x, edge_index, params):
    srcr = edge_index[0].reshape(NW, NBLK, EB)
    dstr = edge_index[1].reshape(NW, NBLK, EB)

    degp = _deg_kernel(srcr)

    w0cat = jnp.concatenate(
        [params['x_z']['W'][0], params['x_r']['W'][0], params['x_h']['W'][0]], axis=1)
    w1cat = jnp.concatenate(
        [params['x_z']['W'][1], params['x_r']['W'][1], params['x_h']['W'][1]], axis=1)
    bcat = jnp.concatenate(
        [params['x_z']['b'] + params['h_z']['b'],
         params['x_r']['b'] + params['h_r']['b'],
         params['x_h']['b'] + params['h_h']['b']]).reshape(1, 3 * HID)

    xp0, xpre = _prep(x, w0cat, w1cat, bcat, degp)

    sxp = _sc_pass4(xpre[0], xpre[1], xpre[2], xpre[3], srcr, dstr)

    h, hp = _t0(xp0, sxp[0], degp)

    for t in range(1, T):
        (shp,) = _sc_pass1(hp, srcr, dstr)
        z, g, gp = _gates(t, xp0, sxp[t], h, shp, degp,
                          params['h_z']['W'][0], params['h_r']['W'][0],
                          params['h_z']['W'][1], params['h_r']['W'][1])
        (sgp,) = _sc_pass1(gp, srcr, dstr)
        if t < T - 1:
            h, hp = _update(t, xp0, sxp[t], z, g, sgp, h, degp,
                            params['h_h']['W'][0], params['h_h']['W'][1])
        else:
            h, x_pred = _final(t, xp0, sxp[t], z, g, sgp, h, degp,
                               params['h_h']['W'][0], params['h_h']['W'][1],
                               params['dec_W1'], params['dec_b1'].reshape(1, HID),
                               params['dec_W2'], params['dec_b2'].reshape(1, F_IN))

    return (x_pred, h)


# speedup vs baseline: 1.0282x; 1.0181x over previous
import functools
import jax
import jax.numpy as jnp
from jax import lax
from jax.experimental import pallas as pl
from jax.experimental.pallas import tpu as pltpu
from jax.experimental.pallas import tpu_sc as plsc

N = 10000
E = 320000
F_IN = 128
HID = 32
T = 4

BLK = 2000
NB = N // BLK

NW = 32
EPT = E // NW
EB = 200
NBLK = EPT // EB
RB = 624
RT = N - 16 * RB
ZC = 48


def _sc_mesh():
    return plsc.VectorSubcoreMesh(core_axis_name="c", subcore_axis_name="s")


def _deg_kernel(srcr):
    @functools.partial(
        pl.kernel,
        mesh=_sc_mesh(),
        out_type=jax.ShapeDtypeStruct((2, N, 16), jnp.float32),
        scratch_types=[
            pltpu.VMEM((NBLK, EB), jnp.int32),
            pltpu.VMEM((EB, 16), jnp.float32),
            pltpu.VMEM((ZC, 16), jnp.float32),
            pltpu.VMEM_SHARED((N, 16), jnp.float32),
        ],
        compiler_params=pltpu.CompilerParams(use_tc_tiling_on_sc=False),
    )
    def k(srcr_hbm, out_hbm, src_v, ones_v, zrow_v, acc):
        c = lax.axis_index("c")
        s = lax.axis_index("s")
        wid = c * 16 + s
        pltpu.sync_copy(srcr_hbm.at[wid], src_v)

        def fill(i, carry):
            ones_v[i, :] = jnp.ones((16,), jnp.float32)
            return carry
        lax.fori_loop(0, EB, fill, 0)

        def zfill(i, carry):
            zrow_v[i, :] = jnp.zeros((16,), jnp.float32)
            return carry
        lax.fori_loop(0, ZC, zfill, 0)
        for kk in range(RB // ZC):
            pltpu.sync_copy(zrow_v, acc.at[pl.ds(s * RB + kk * ZC, ZC)])

        @pl.when(s == 15)
        def _():
            pltpu.sync_copy(zrow_v.at[pl.ds(0, RT)], acc.at[pl.ds(16 * RB, RT)])
        plsc.subcore_barrier()

        def body(j, carry):
            pltpu.sync_copy(ones_v, acc.at[src_v.at[j]], add=True)
            return carry
        lax.fori_loop(0, NBLK, body, 0)
        plsc.subcore_barrier()
        pltpu.sync_copy(acc.at[pl.ds(s * RB, RB)],
                        out_hbm.at[c, pl.ds(s * RB, RB)])

        @pl.when(s == 15)
        def _():
            pltpu.sync_copy(acc.at[pl.ds(16 * RB, RT)],
                            out_hbm.at[c, pl.ds(16 * RB, RT)])

    return k(srcr)


def _make_sc_pass(W, ntab, nbuf):
    @functools.partial(
        pl.kernel,
        mesh=_sc_mesh(),
        out_type=[jax.ShapeDtypeStruct((2, N, W), jnp.float32)] * ntab,
        scratch_types=[
            pltpu.VMEM((NBLK, EB), jnp.int32),
            pltpu.VMEM((NBLK, EB), jnp.int32),
            pltpu.VMEM((nbuf, EB, W), jnp.float32),
            pltpu.VMEM((ZC, W), jnp.float32),
            pltpu.VMEM_SHARED((N, W), jnp.float32),
        ] + [pltpu.SemaphoreType.DMA] * nbuf,
        compiler_params=pltpu.CompilerParams(use_tc_tiling_on_sc=False),
    )
    def k(*args):
        tabs_hbm = args[0]
        srcr, dstr = args[1], args[2]
        outs = args[3:3 + ntab]
        src_v, dst_v, bufs, zrow_v, acc = args[3 + ntab:8 + ntab]
        sems = args[8 + ntab:]
        c = lax.axis_index("c")
        s = lax.axis_index("s")
        wid = c * 16 + s
        pltpu.sync_copy(srcr.at[wid], src_v)
        pltpu.sync_copy(dstr.at[wid], dst_v)

        def zfill(i, carry):
            for kk in range(W // 16):
                zrow_v[i, pl.ds(kk * 16, 16)] = jnp.zeros((16,), jnp.float32)
            return carry
        lax.fori_loop(0, ZC, zfill, 0)

        for ti in range(ntab):
            tab = tabs_hbm.at[ti] if ntab > 1 else tabs_hbm
            for b in range(nbuf):
                pltpu.async_copy(tab.at[src_v.at[b]], bufs.at[b], sems[b])

            for kk in range(RB // ZC):
                pltpu.sync_copy(zrow_v, acc.at[pl.ds(s * RB + kk * ZC, ZC)])

            @pl.when(s == 15)
            def _():
                pltpu.sync_copy(zrow_v.at[pl.ds(0, RT)],
                                acc.at[pl.ds(16 * RB, RT)])
            plsc.subcore_barrier()

            def body(g, carry):
                for b in range(nbuf):
                    j = g * nbuf + b
                    pltpu.make_async_copy(
                        tab.at[pl.ds(0, EB)], bufs.at[b], sems[b]).wait()
                    pltpu.sync_copy(bufs.at[b], acc.at[dst_v.at[j]], add=True)

                    @pl.when(g < NBLK // nbuf - 1)
                    def _(b=b, j=j):
                        pltpu.async_copy(tab.at[src_v.at[j + nbuf]],
                                         bufs.at[b], sems[b])
                return carry
            lax.fori_loop(0, NBLK // nbuf, body, 0)
            plsc.subcore_barrier()
            pltpu.sync_copy(acc.at[pl.ds(s * RB, RB)],
                            outs[ti].at[c, pl.ds(s * RB, RB)])

            @pl.when(s == 15)
            def _():
                pltpu.sync_copy(acc.at[pl.ds(16 * RB, RT)],
                                outs[ti].at[c, pl.ds(16 * RB, RT)])
            plsc.subcore_barrier()

    return k


_sc_pass4 = _make_sc_pass(3 * HID, 4, 2)
_sc_pass1 = _make_sc_pass(HID, 1, 5)


def _dinv_of(degp_ref):
    deg = degp_ref[0, :, 0:1] + degp_ref[1, :, 0:1]
    return jnp.where(deg > 0, lax.rsqrt(deg), 0.0)


def _prep_body(x_ref, w0_ref, w1_ref, b_ref, degp_ref, xp0_ref, xpre_ref):
    xb = x_ref[0]
    dinv = _dinv_of(degp_ref)
    xp0_ref[0] = jnp.dot(xb, w0_ref[...], preferred_element_type=jnp.float32) + b_ref[...]
    xpre_ref[0] = dinv * jnp.dot(xb, w1_ref[...], preferred_element_type=jnp.float32)


def _prep(x, w0cat, w1cat, bcat, degp):
    return pl.pallas_call(
        _prep_body,
        grid=(T, NB),
        in_specs=[
            pl.BlockSpec((1, BLK, F_IN), lambda t, i: (t, i, 0)),
            pl.BlockSpec((F_IN, 3 * HID), lambda t, i: (0, 0)),
            pl.BlockSpec((F_IN, 3 * HID), lambda t, i: (0, 0)),
            pl.BlockSpec((1, 3 * HID), lambda t, i: (0, 0)),
            pl.BlockSpec((2, BLK, 16), lambda t, i: (0, i, 0)),
        ],
        out_specs=[
            pl.BlockSpec((1, BLK, 3 * HID), lambda t, i: (t, i, 0)),
            pl.BlockSpec((1, BLK, 3 * HID), lambda t, i: (t, i, 0)),
        ],
        out_shape=[
            jax.ShapeDtypeStruct((T, N, 3 * HID), jnp.float32),
            jax.ShapeDtypeStruct((T, N, 3 * HID), jnp.float32),
        ],
    )(x, w0cat, w1cat, bcat, degp)


def _t0_body(xp0_ref, sx_ref, degp_ref, h_ref, hp_ref):
    dinv = _dinv_of(degp_ref)
    pre = xp0_ref[0] - dinv * (sx_ref[0] + sx_ref[1])
    z = jax.nn.sigmoid(pre[:, 0:HID])
    ht = jnp.tanh(pre[:, 2 * HID:3 * HID])
    h = (1.0 - z) * ht
    h_ref[...] = h
    hp_ref[...] = dinv * h


def _t0(xp0, sx, degp):
    return pl.pallas_call(
        _t0_body,
        grid=(NB,),
        in_specs=[
            pl.BlockSpec((1, BLK, 3 * HID), lambda i: (0, i, 0)),
            pl.BlockSpec((2, BLK, 3 * HID), lambda i: (0, i, 0)),
            pl.BlockSpec((2, BLK, 16), lambda i: (0, i, 0)),
        ],
        out_specs=[
            pl.BlockSpec((BLK, HID), lambda i: (i, 0)),
            pl.BlockSpec((BLK, HID), lambda i: (i, 0)),
        ],
        out_shape=[
            jax.ShapeDtypeStruct((N, HID), jnp.float32),
            jax.ShapeDtypeStruct((N, HID), jnp.float32),
        ],
    )(xp0, sx, degp)


def _gates_body(xp0_ref, sx_ref, h_ref, sh_ref, degp_ref,
                wz0_ref, wr0_ref, wz1_ref, wr1_ref,
                z_ref, g_ref, gp_ref):
    dinv = _dinv_of(degp_ref)
    h = h_ref[...]
    lh = -dinv * (sh_ref[0] + sh_ref[1])
    xp = xp0_ref[0] - dinv * (sx_ref[0] + sx_ref[1])
    z = jax.nn.sigmoid(
        xp[:, 0:HID]
        + jnp.dot(h, wz0_ref[...], preferred_element_type=jnp.float32)
        + jnp.dot(lh, wz1_ref[...], preferred_element_type=jnp.float32))
    r = jax.nn.sigmoid(
        xp[:, HID:2 * HID]
        + jnp.dot(h, wr0_ref[...], preferred_element_type=jnp.float32)
        + jnp.dot(lh, wr1_ref[...], preferred_element_type=jnp.float32))
    g = h * r
    z_ref[...] = z
    g_ref[...] = g
    gp_ref[...] = dinv * g


def _gates(t, xp0, sx, h, sh, degp, wz0, wr0, wz1, wr1):
    return pl.pallas_call(
        _gates_body,
        grid=(NB,),
        in_specs=[
            pl.BlockSpec((1, BLK, 3 * HID), lambda i, t=t: (t, i, 0)),
            pl.BlockSpec((2, BLK, 3 * HID), lambda i: (0, i, 0)),
            pl.BlockSpec((BLK, HID), lambda i: (i, 0)),
            pl.BlockSpec((2, BLK, HID), lambda i: (0, i, 0)),
            pl.BlockSpec((2, BLK, 16), lambda i: (0, i, 0)),
            pl.BlockSpec((HID, HID), lambda i: (0, 0)),
            pl.BlockSpec((HID, HID), lambda i: (0, 0)),
            pl.BlockSpec((HID, HID), lambda i: (0, 0)),
            pl.BlockSpec((HID, HID), lambda i: (0, 0)),
        ],
        out_specs=[
            pl.BlockSpec((BLK, HID), lambda i: (i, 0)),
            pl.BlockSpec((BLK, HID), lambda i: (i, 0)),
            pl.BlockSpec((BLK, HID), lambda i: (i, 0)),
        ],
        out_shape=[
            jax.ShapeDtypeStruct((N, HID), jnp.float32),
            jax.ShapeDtypeStruct((N, HID), jnp.float32),
            jax.ShapeDtypeStruct((N, HID), jnp.float32),
        ],
    )(xp0, sx, h, sh, degp, wz0, wr0, wz1, wr1)


def _update_body(xp0_ref, sx_ref, z_ref, g_ref, sg_ref, h_ref, degp_ref,
                 wh0_ref, wh1_ref, hn_ref, hp_ref):
    dinv = _dinv_of(degp_ref)
    g = g_ref[...]
    lg = -dinv * (sg_ref[0] + sg_ref[1])
    xh = (xp0_ref[0, :, 2 * HID:3 * HID]
          - dinv * (sx_ref[0, :, 2 * HID:3 * HID] + sx_ref[1, :, 2 * HID:3 * HID]))
    ht = jnp.tanh(
        xh
        + jnp.dot(g, wh0_ref[...], preferred_element_type=jnp.float32)
        + jnp.dot(lg, wh1_ref[...], preferred_element_type=jnp.float32))
    z = z_ref[...]
    hn = z * h_ref[...] + (1.0 - z) * ht
    hn_ref[...] = hn
    hp_ref[...] = dinv * hn


def _update(t, xp0, sx, z, g, sg, h, degp, wh0, wh1):
    return pl.pallas_call(
        _update_body,
        grid=(NB,),
        in_specs=[
            pl.BlockSpec((1, BLK, 3 * HID), lambda i, t=t: (t, i, 0)),
            pl.BlockSpec((2, BLK, 3 * HID), lambda i: (0, i, 0)),
            pl.BlockSpec((BLK, HID), lambda i: (i, 0)),
            pl.BlockSpec((BLK, HID), lambda i: (i, 0)),
            pl.BlockSpec((2, BLK, HID), lambda i: (0, i, 0)),
            pl.BlockSpec((BLK, HID), lambda i: (i, 0)),
            pl.BlockSpec((2, BLK, 16), lambda i: (0, i, 0)),
            pl.BlockSpec((HID, HID), lambda i: (0, 0)),
            pl.BlockSpec((HID, HID), lambda i: (0, 0)),
        ],
        out_specs=[
            pl.BlockSpec((BLK, HID), lambda i: (i, 0)),
            pl.BlockSpec((BLK, HID), lambda i: (i, 0)),
        ],
        out_shape=[
            jax.ShapeDtypeStruct((N, HID), jnp.float32),
            jax.ShapeDtypeStruct((N, HID), jnp.float32),
        ],
    )(xp0, sx, z, g, sg, h, degp, wh0, wh1)


def _final_body(xp0_ref, sx_ref, z_ref, g_ref, sg_ref, h_ref, degp_ref,
                wh0_ref, wh1_ref, w1_ref, b1_ref, w2_ref, b2_ref,
                hn_ref, out_ref):
    dinv = _dinv_of(degp_ref)
    g = g_ref[...]
    lg = -dinv * (sg_ref[0] + sg_ref[1])
    xh = (xp0_ref[0, :, 2 * HID:3 * HID]
          - dinv * (sx_ref[0, :, 2 * HID:3 * HID] + sx_ref[1, :, 2 * HID:3 * HID]))
    ht = jnp.tanh(
        xh
        + jnp.dot(g, wh0_ref[...], preferred_element_type=jnp.float32)
        + jnp.dot(lg, wh1_ref[...], preferred_element_type=jnp.float32))
    z = z_ref[...]
    hn = z * h_ref[...] + (1.0 - z) * ht
    hn_ref[...] = hn
    h1 = jax.nn.relu(
        jnp.dot(hn, w1_ref[...], preferred_element_type=jnp.float32)
        + b1_ref[...])
    out_ref[...] = jnp.dot(h1, w2_ref[...], preferred_element_type=jnp.float32) + b2_ref[...]


def _final(t, xp0, sx, z, g, sg, h, degp, wh0, wh1, w1, b1, w2, b2):
    return pl.pallas_call(
        _final_body,
        grid=(NB,),
        in_specs=[
            pl.BlockSpec((1, BLK, 3 * HID), lambda i, t=t: (t, i, 0)),
            pl.BlockSpec((2, BLK, 3 * HID), lambda i: (0, i, 0)),
            pl.BlockSpec((BLK, HID), lambda i: (i, 0)),
            pl.BlockSpec((BLK, HID), lambda i: (i, 0)),
            pl.BlockSpec((2, BLK, HID), lambda i: (0, i, 0)),
            pl.BlockSpec((BLK, HID), lambda i: (i, 0)),
            pl.BlockSpec((2, BLK, 16), lambda i: (0, i, 0)),
            pl.BlockSpec((HID, HID), lambda i: (0, 0)),
            pl.BlockSpec((HID, HID), lambda i: (0, 0)),
            pl.BlockSpec((HID, HID), lambda i: (0, 0)),
            pl.BlockSpec((1, HID), lambda i: (0, 0)),
            pl.BlockSpec((HID, F_IN), lambda i: (0, 0)),
            pl.BlockSpec((1, F_IN), lambda i: (0, 0)),
        ],
        out_specs=[
            pl.BlockSpec((BLK, HID), lambda i: (i, 0)),
            pl.BlockSpec((BLK, F_IN), lambda i: (i, 0)),
        ],
        out_shape=[
            jax.ShapeDtypeStruct((N, HID), jnp.float32),
            jax.ShapeDtypeStruct((N, F_IN), jnp.float32),
        ],
    )(xp0, sx, z, g, sg, h, degp, wh0, wh1, w1, b1, w2, b2)


def kernel(x, edge_index, params):
    srcr = edge_index[0].reshape(NW, NBLK, EB)
    dstr = edge_index[1].reshape(NW, NBLK, EB)

    degp = _deg_kernel(srcr)

    w0cat = jnp.concatenate(
        [params['x_z']['W'][0], params['x_r']['W'][0], params['x_h']['W'][0]], axis=1)
    w1cat = jnp.concatenate(
        [params['x_z']['W'][1], params['x_r']['W'][1], params['x_h']['W'][1]], axis=1)
    bcat = jnp.concatenate(
        [params['x_z']['b'] + params['h_z']['b'],
         params['x_r']['b'] + params['h_r']['b'],
         params['x_h']['b'] + params['h_h']['b']]).reshape(1, 3 * HID)

    xp0, xpre = _prep(x, w0cat, w1cat, bcat, degp)

    sxp = _sc_pass4(xpre, srcr, dstr)

    h, hp = _t0(xp0, sxp[0], degp)

    for t in range(1, T):
        (shp,) = _sc_pass1(hp, srcr, dstr)
        z, g, gp = _gates(t, xp0, sxp[t], h, shp, degp,
                          params['h_z']['W'][0], params['h_r']['W'][0],
                          params['h_z']['W'][1], params['h_r']['W'][1])
        (sgp,) = _sc_pass1(gp, srcr, dstr)
        if t < T - 1:
            h, hp = _update(t, xp0, sxp[t], z, g, sgp, h, degp,
                            params['h_h']['W'][0], params['h_h']['W'][1])
        else:
            h, x_pred = _final(t, xp0, sxp[t], z, g, sgp, h, degp,
                               params['h_h']['W'][0], params['h_h']['W'][1],
                               params['dec_W1'], params['dec_b1'].reshape(1, HID),
                               params['dec_W2'], params['dec_b2'].reshape(1, F_IN))

    return (x_pred, h)


# speedup vs baseline: 1.0406x; 1.0121x over previous
import functools
import jax
import jax.numpy as jnp
from jax import lax
from jax.experimental import pallas as pl
from jax.experimental.pallas import tpu as pltpu
from jax.experimental.pallas import tpu_sc as plsc

N = 10000
E = 320000
F_IN = 128
HID = 32
T = 4

BLK = 2000
NB = N // BLK

NW = 32
EPT = E // NW
EB = 200
NBLK = EPT // EB
RB = 624
RT = N - 16 * RB
ZC = 48


def _sc_mesh():
    return plsc.VectorSubcoreMesh(core_axis_name="c", subcore_axis_name="s")


def _deg_kernel(srcr):
    @functools.partial(
        pl.kernel,
        mesh=_sc_mesh(),
        out_type=jax.ShapeDtypeStruct((2, N, 16), jnp.float32),
        scratch_types=[
            pltpu.VMEM((NBLK, EB), jnp.int32),
            pltpu.VMEM((EB, 16), jnp.float32),
            pltpu.VMEM((ZC, 16), jnp.float32),
            pltpu.VMEM_SHARED((N, 16), jnp.float32),
        ],
        compiler_params=pltpu.CompilerParams(use_tc_tiling_on_sc=False),
    )
    def k(srcr_hbm, out_hbm, src_v, ones_v, zrow_v, acc):
        c = lax.axis_index("c")
        s = lax.axis_index("s")
        wid = c * 16 + s
        pltpu.sync_copy(srcr_hbm.at[wid], src_v)

        def fill(i, carry):
            ones_v[i, :] = jnp.ones((16,), jnp.float32)
            return carry
        lax.fori_loop(0, EB, fill, 0)

        def zfill(i, carry):
            zrow_v[i, :] = jnp.zeros((16,), jnp.float32)
            return carry
        lax.fori_loop(0, ZC, zfill, 0)
        for kk in range(RB // ZC):
            pltpu.sync_copy(zrow_v, acc.at[pl.ds(s * RB + kk * ZC, ZC)])

        @pl.when(s == 15)
        def _():
            pltpu.sync_copy(zrow_v.at[pl.ds(0, RT)], acc.at[pl.ds(16 * RB, RT)])
        plsc.subcore_barrier()

        def body(j, carry):
            pltpu.sync_copy(ones_v, acc.at[src_v.at[j]], add=True)
            return carry
        lax.fori_loop(0, NBLK, body, 0)
        plsc.subcore_barrier()
        pltpu.sync_copy(acc.at[pl.ds(s * RB, RB)],
                        out_hbm.at[c, pl.ds(s * RB, RB)])

        @pl.when(s == 15)
        def _():
            pltpu.sync_copy(acc.at[pl.ds(16 * RB, RT)],
                            out_hbm.at[c, pl.ds(16 * RB, RT)])

    return k(srcr)


def _make_sc_pass(W, ntab, nbuf, off=None):
    @functools.partial(
        pl.kernel,
        mesh=_sc_mesh(),
        out_type=[jax.ShapeDtypeStruct((2, N, W), jnp.float32)] * ntab,
        scratch_types=[
            pltpu.VMEM((NBLK, EB), jnp.int32),
            pltpu.VMEM((NBLK, EB), jnp.int32),
            pltpu.VMEM((nbuf, EB, W), jnp.float32),
            pltpu.VMEM((ZC, W), jnp.float32),
            pltpu.VMEM_SHARED((N, W), jnp.float32),
        ] + [pltpu.SemaphoreType.DMA] * nbuf,
        compiler_params=pltpu.CompilerParams(use_tc_tiling_on_sc=False),
    )
    def k(*args):
        tabs_hbm = args[0]
        srcr, dstr = args[1], args[2]
        outs = args[3:3 + ntab]
        src_v, dst_v, bufs, zrow_v, acc = args[3 + ntab:8 + ntab]
        sems = args[8 + ntab:]
        c = lax.axis_index("c")
        s = lax.axis_index("s")
        wid = c * 16 + s
        pltpu.sync_copy(srcr.at[wid], src_v)
        pltpu.sync_copy(dstr.at[wid], dst_v)

        def zfill(i, carry):
            for kk in range(W // 16):
                zrow_v[i, pl.ds(kk * 16, 16)] = jnp.zeros((16,), jnp.float32)
            return carry
        lax.fori_loop(0, ZC, zfill, 0)

        for ti in range(ntab):
            tab = tabs_hbm if off is None else tabs_hbm.at[off + ti]
            for b in range(nbuf):
                pltpu.async_copy(tab.at[src_v.at[b]], bufs.at[b], sems[b])

            for kk in range(RB // ZC):
                pltpu.sync_copy(zrow_v, acc.at[pl.ds(s * RB + kk * ZC, ZC)])

            @pl.when(s == 15)
            def _():
                pltpu.sync_copy(zrow_v.at[pl.ds(0, RT)],
                                acc.at[pl.ds(16 * RB, RT)])
            plsc.subcore_barrier()

            def body(g, carry):
                for b in range(nbuf):
                    j = g * nbuf + b
                    pltpu.make_async_copy(
                        tab.at[pl.ds(0, EB)], bufs.at[b], sems[b]).wait()
                    pltpu.sync_copy(bufs.at[b], acc.at[dst_v.at[j]], add=True)

                    @pl.when(g < NBLK // nbuf - 1)
                    def _(b=b, j=j):
                        pltpu.async_copy(tab.at[src_v.at[j + nbuf]],
                                         bufs.at[b], sems[b])
                return carry
            lax.fori_loop(0, NBLK // nbuf, body, 0)
            plsc.subcore_barrier()
            pltpu.sync_copy(acc.at[pl.ds(s * RB, RB)],
                            outs[ti].at[c, pl.ds(s * RB, RB)])

            @pl.when(s == 15)
            def _():
                pltpu.sync_copy(acc.at[pl.ds(16 * RB, RT)],
                                outs[ti].at[c, pl.ds(16 * RB, RT)])
            plsc.subcore_barrier()

    return k


_sc_passA = _make_sc_pass(3 * HID, 1, 2, off=0)
_sc_passB = _make_sc_pass(3 * HID, 3, 2, off=1)
_sc_pass1 = _make_sc_pass(HID, 1, 5)


def _dinv_of(degp_ref):
    deg = degp_ref[0, :, 0:1] + degp_ref[1, :, 0:1]
    return jnp.where(deg > 0, lax.rsqrt(deg), 0.0)


def _prep_body(x_ref, w0_ref, w1_ref, b_ref, degp_ref, xp0_ref, xpre_ref):
    xb = x_ref[0]
    dinv = _dinv_of(degp_ref)
    xp0_ref[0] = jnp.dot(xb, w0_ref[...], preferred_element_type=jnp.float32) + b_ref[...]
    xpre_ref[0] = dinv * jnp.dot(xb, w1_ref[...], preferred_element_type=jnp.float32)


def _prep(x, w0cat, w1cat, bcat, degp):
    return pl.pallas_call(
        _prep_body,
        grid=(T, NB),
        in_specs=[
            pl.BlockSpec((1, BLK, F_IN), lambda t, i: (t, i, 0)),
            pl.BlockSpec((F_IN, 3 * HID), lambda t, i: (0, 0)),
            pl.BlockSpec((F_IN, 3 * HID), lambda t, i: (0, 0)),
            pl.BlockSpec((1, 3 * HID), lambda t, i: (0, 0)),
            pl.BlockSpec((2, BLK, 16), lambda t, i: (0, i, 0)),
        ],
        out_specs=[
            pl.BlockSpec((1, BLK, 3 * HID), lambda t, i: (t, i, 0)),
            pl.BlockSpec((1, BLK, 3 * HID), lambda t, i: (t, i, 0)),
        ],
        out_shape=[
            jax.ShapeDtypeStruct((T, N, 3 * HID), jnp.float32),
            jax.ShapeDtypeStruct((T, N, 3 * HID), jnp.float32),
        ],
    )(x, w0cat, w1cat, bcat, degp)


def _t0_body(xp0_ref, sx_ref, degp_ref, h_ref, hp_ref):
    dinv = _dinv_of(degp_ref)
    pre = xp0_ref[0] - dinv * (sx_ref[0] + sx_ref[1])
    z = jax.nn.sigmoid(pre[:, 0:HID])
    ht = jnp.tanh(pre[:, 2 * HID:3 * HID])
    h = (1.0 - z) * ht
    h_ref[...] = h
    hp_ref[...] = dinv * h


def _t0(xp0, sx, degp):
    return pl.pallas_call(
        _t0_body,
        grid=(NB,),
        in_specs=[
            pl.BlockSpec((1, BLK, 3 * HID), lambda i: (0, i, 0)),
            pl.BlockSpec((2, BLK, 3 * HID), lambda i: (0, i, 0)),
            pl.BlockSpec((2, BLK, 16), lambda i: (0, i, 0)),
        ],
        out_specs=[
            pl.BlockSpec((BLK, HID), lambda i: (i, 0)),
            pl.BlockSpec((BLK, HID), lambda i: (i, 0)),
        ],
        out_shape=[
            jax.ShapeDtypeStruct((N, HID), jnp.float32),
            jax.ShapeDtypeStruct((N, HID), jnp.float32),
        ],
    )(xp0, sx, degp)


def _gates_body(xp0_ref, sx_ref, h_ref, sh_ref, degp_ref,
                wz0_ref, wr0_ref, wz1_ref, wr1_ref,
                z_ref, g_ref, gp_ref):
    dinv = _dinv_of(degp_ref)
    h = h_ref[...]
    lh = -dinv * (sh_ref[0] + sh_ref[1])
    xp = xp0_ref[0] - dinv * (sx_ref[0] + sx_ref[1])
    z = jax.nn.sigmoid(
        xp[:, 0:HID]
        + jnp.dot(h, wz0_ref[...], preferred_element_type=jnp.float32)
        + jnp.dot(lh, wz1_ref[...], preferred_element_type=jnp.float32))
    r = jax.nn.sigmoid(
        xp[:, HID:2 * HID]
        + jnp.dot(h, wr0_ref[...], preferred_element_type=jnp.float32)
        + jnp.dot(lh, wr1_ref[...], preferred_element_type=jnp.float32))
    g = h * r
    z_ref[...] = z
    g_ref[...] = g
    gp_ref[...] = dinv * g


def _gates(t, xp0, sx, h, sh, degp, wz0, wr0, wz1, wr1):
    return pl.pallas_call(
        _gates_body,
        grid=(NB,),
        in_specs=[
            pl.BlockSpec((1, BLK, 3 * HID), lambda i, t=t: (t, i, 0)),
            pl.BlockSpec((2, BLK, 3 * HID), lambda i: (0, i, 0)),
            pl.BlockSpec((BLK, HID), lambda i: (i, 0)),
            pl.BlockSpec((2, BLK, HID), lambda i: (0, i, 0)),
            pl.BlockSpec((2, BLK, 16), lambda i: (0, i, 0)),
            pl.BlockSpec((HID, HID), lambda i: (0, 0)),
            pl.BlockSpec((HID, HID), lambda i: (0, 0)),
            pl.BlockSpec((HID, HID), lambda i: (0, 0)),
            pl.BlockSpec((HID, HID), lambda i: (0, 0)),
        ],
        out_specs=[
            pl.BlockSpec((BLK, HID), lambda i: (i, 0)),
            pl.BlockSpec((BLK, HID), lambda i: (i, 0)),
            pl.BlockSpec((BLK, HID), lambda i: (i, 0)),
        ],
        out_shape=[
            jax.ShapeDtypeStruct((N, HID), jnp.float32),
            jax.ShapeDtypeStruct((N, HID), jnp.float32),
            jax.ShapeDtypeStruct((N, HID), jnp.float32),
        ],
    )(xp0, sx, h, sh, degp, wz0, wr0, wz1, wr1)


def _update_body(xp0_ref, sx_ref, z_ref, g_ref, sg_ref, h_ref, degp_ref,
                 wh0_ref, wh1_ref, hn_ref, hp_ref):
    dinv = _dinv_of(degp_ref)
    g = g_ref[...]
    lg = -dinv * (sg_ref[0] + sg_ref[1])
    xh = (xp0_ref[0, :, 2 * HID:3 * HID]
          - dinv * (sx_ref[0, :, 2 * HID:3 * HID] + sx_ref[1, :, 2 * HID:3 * HID]))
    ht = jnp.tanh(
        xh
        + jnp.dot(g, wh0_ref[...], preferred_element_type=jnp.float32)
        + jnp.dot(lg, wh1_ref[...], preferred_element_type=jnp.float32))
    z = z_ref[...]
    hn = z * h_ref[...] + (1.0 - z) * ht
    hn_ref[...] = hn
    hp_ref[...] = dinv * hn


def _update(t, xp0, sx, z, g, sg, h, degp, wh0, wh1):
    return pl.pallas_call(
        _update_body,
        grid=(NB,),
        in_specs=[
            pl.BlockSpec((1, BLK, 3 * HID), lambda i, t=t: (t, i, 0)),
            pl.BlockSpec((2, BLK, 3 * HID), lambda i: (0, i, 0)),
            pl.BlockSpec((BLK, HID), lambda i: (i, 0)),
            pl.BlockSpec((BLK, HID), lambda i: (i, 0)),
            pl.BlockSpec((2, BLK, HID), lambda i: (0, i, 0)),
            pl.BlockSpec((BLK, HID), lambda i: (i, 0)),
            pl.BlockSpec((2, BLK, 16), lambda i: (0, i, 0)),
            pl.BlockSpec((HID, HID), lambda i: (0, 0)),
            pl.BlockSpec((HID, HID), lambda i: (0, 0)),
        ],
        out_specs=[
            pl.BlockSpec((BLK, HID), lambda i: (i, 0)),
            pl.BlockSpec((BLK, HID), lambda i: (i, 0)),
        ],
        out_shape=[
            jax.ShapeDtypeStruct((N, HID), jnp.float32),
            jax.ShapeDtypeStruct((N, HID), jnp.float32),
        ],
    )(xp0, sx, z, g, sg, h, degp, wh0, wh1)


def _final_body(xp0_ref, sx_ref, z_ref, g_ref, sg_ref, h_ref, degp_ref,
                wh0_ref, wh1_ref, w1_ref, b1_ref, w2_ref, b2_ref,
                hn_ref, out_ref):
    dinv = _dinv_of(degp_ref)
    g = g_ref[...]
    lg = -dinv * (sg_ref[0] + sg_ref[1])
    xh = (xp0_ref[0, :, 2 * HID:3 * HID]
          - dinv * (sx_ref[0, :, 2 * HID:3 * HID] + sx_ref[1, :, 2 * HID:3 * HID]))
    ht = jnp.tanh(
        xh
        + jnp.dot(g, wh0_ref[...], preferred_element_type=jnp.float32)
        + jnp.dot(lg, wh1_ref[...], preferred_element_type=jnp.float32))
    z = z_ref[...]
    hn = z * h_ref[...] + (1.0 - z) * ht
    hn_ref[...] = hn
    h1 = jax.nn.relu(
        jnp.dot(hn, w1_ref[...], preferred_element_type=jnp.float32)
        + b1_ref[...])
    out_ref[...] = jnp.dot(h1, w2_ref[...], preferred_element_type=jnp.float32) + b2_ref[...]


def _final(t, xp0, sx, z, g, sg, h, degp, wh0, wh1, w1, b1, w2, b2):
    return pl.pallas_call(
        _final_body,
        grid=(NB,),
        in_specs=[
            pl.BlockSpec((1, BLK, 3 * HID), lambda i, t=t: (t, i, 0)),
            pl.BlockSpec((2, BLK, 3 * HID), lambda i: (0, i, 0)),
            pl.BlockSpec((BLK, HID), lambda i: (i, 0)),
            pl.BlockSpec((BLK, HID), lambda i: (i, 0)),
            pl.BlockSpec((2, BLK, HID), lambda i: (0, i, 0)),
            pl.BlockSpec((BLK, HID), lambda i: (i, 0)),
            pl.BlockSpec((2, BLK, 16), lambda i: (0, i, 0)),
            pl.BlockSpec((HID, HID), lambda i: (0, 0)),
            pl.BlockSpec((HID, HID), lambda i: (0, 0)),
            pl.BlockSpec((HID, HID), lambda i: (0, 0)),
            pl.BlockSpec((1, HID), lambda i: (0, 0)),
            pl.BlockSpec((HID, F_IN), lambda i: (0, 0)),
            pl.BlockSpec((1, F_IN), lambda i: (0, 0)),
        ],
        out_specs=[
            pl.BlockSpec((BLK, HID), lambda i: (i, 0)),
            pl.BlockSpec((BLK, F_IN), lambda i: (i, 0)),
        ],
        out_shape=[
            jax.ShapeDtypeStruct((N, HID), jnp.float32),
            jax.ShapeDtypeStruct((N, F_IN), jnp.float32),
        ],
    )(xp0, sx, z, g, sg, h, degp, wh0, wh1, w1, b1, w2, b2)


def kernel(x, edge_index, params):
    srcr = edge_index[0].reshape(NW, NBLK, EB)
    dstr = edge_index[1].reshape(NW, NBLK, EB)

    degp = _deg_kernel(srcr)

    w0cat = jnp.concatenate(
        [params['x_z']['W'][0], params['x_r']['W'][0], params['x_h']['W'][0]], axis=1)
    w1cat = jnp.concatenate(
        [params['x_z']['W'][1], params['x_r']['W'][1], params['x_h']['W'][1]], axis=1)
    bcat = jnp.concatenate(
        [params['x_z']['b'] + params['h_z']['b'],
         params['x_r']['b'] + params['h_r']['b'],
         params['x_h']['b'] + params['h_h']['b']]).reshape(1, 3 * HID)

    xp0, xpre = _prep(x, w0cat, w1cat, bcat, degp)

    (sxp0,) = _sc_passA(xpre, srcr, dstr)
    sxpB = _sc_passB(xpre, srcr, dstr)
    sxp = [sxp0] + list(sxpB)

    h, hp = _t0(xp0, sxp[0], degp)

    for t in range(1, T):
        (shp,) = _sc_pass1(hp, srcr, dstr)
        z, g, gp = _gates(t, xp0, sxp[t], h, shp, degp,
                          params['h_z']['W'][0], params['h_r']['W'][0],
                          params['h_z']['W'][1], params['h_r']['W'][1])
        (sgp,) = _sc_pass1(gp, srcr, dstr)
        if t < T - 1:
            h, hp = _update(t, xp0, sxp[t], z, g, sgp, h, degp,
                            params['h_h']['W'][0], params['h_h']['W'][1])
        else:
            h, x_pred = _final(t, xp0, sxp[t], z, g, sgp, h, degp,
                               params['h_h']['W'][0], params['h_h']['W'][1],
                               params['dec_W1'], params['dec_b1'].reshape(1, HID),
                               params['dec_W2'], params['dec_b2'].reshape(1, F_IN))

    return (x_pred, h)


# speedup vs baseline: 1.0708x; 1.0289x over previous
import functools
import jax
import jax.numpy as jnp
from jax import lax
from jax.experimental import pallas as pl
from jax.experimental.pallas import tpu as pltpu
from jax.experimental.pallas import tpu_sc as plsc

N = 10000
E = 320000
F_IN = 128
HID = 32
T = 4

BLK = 2000
NB = N // BLK

NW = 32
EPT = E // NW
EB = 200
NBLK = EPT // EB
RB = 624
RT = N - 16 * RB
ZC = 48


def _sc_mesh():
    return plsc.VectorSubcoreMesh(core_axis_name="c", subcore_axis_name="s")


def _deg_kernel(srcr):
    @functools.partial(
        pl.kernel,
        mesh=_sc_mesh(),
        out_type=jax.ShapeDtypeStruct((2, N, 16), jnp.float32),
        scratch_types=[
            pltpu.VMEM((NBLK, EB), jnp.int32),
            pltpu.VMEM((EB, 16), jnp.float32),
            pltpu.VMEM((ZC, 16), jnp.float32),
            pltpu.VMEM_SHARED((N, 16), jnp.float32),
        ],
        compiler_params=pltpu.CompilerParams(use_tc_tiling_on_sc=False),
    )
    def k(srcr_hbm, out_hbm, src_v, ones_v, zrow_v, acc):
        c = lax.axis_index("c")
        s = lax.axis_index("s")
        wid = c * 16 + s
        pltpu.sync_copy(srcr_hbm.at[wid], src_v)

        def fill(i, carry):
            ones_v[i, :] = jnp.ones((16,), jnp.float32)
            return carry
        lax.fori_loop(0, EB, fill, 0)

        def zfill(i, carry):
            zrow_v[i, :] = jnp.zeros((16,), jnp.float32)
            return carry
        lax.fori_loop(0, ZC, zfill, 0)
        for kk in range(RB // ZC):
            pltpu.sync_copy(zrow_v, acc.at[pl.ds(s * RB + kk * ZC, ZC)])

        @pl.when(s == 15)
        def _():
            pltpu.sync_copy(zrow_v.at[pl.ds(0, RT)], acc.at[pl.ds(16 * RB, RT)])
        plsc.subcore_barrier()

        def body(j, carry):
            pltpu.sync_copy(ones_v, acc.at[src_v.at[j]], add=True)
            return carry
        lax.fori_loop(0, NBLK, body, 0)
        plsc.subcore_barrier()
        pltpu.sync_copy(acc.at[pl.ds(s * RB, RB)],
                        out_hbm.at[c, pl.ds(s * RB, RB)])

        @pl.when(s == 15)
        def _():
            pltpu.sync_copy(acc.at[pl.ds(16 * RB, RT)],
                            out_hbm.at[c, pl.ds(16 * RB, RT)])

    return k(srcr)


def _make_sc_pass(W, ntab, nbuf):
    @functools.partial(
        pl.kernel,
        mesh=_sc_mesh(),
        out_type=[jax.ShapeDtypeStruct((2, N, W), jnp.float32)] * ntab,
        scratch_types=[
            pltpu.VMEM((NBLK, EB), jnp.int32),
            pltpu.VMEM((NBLK, EB), jnp.int32),
            pltpu.VMEM((nbuf, EB, W), jnp.float32),
            pltpu.VMEM((ZC, W), jnp.float32),
            pltpu.VMEM_SHARED((N, W), jnp.float32),
        ] + [pltpu.SemaphoreType.DMA] * nbuf,
        compiler_params=pltpu.CompilerParams(use_tc_tiling_on_sc=False),
    )
    def k(*args):
        tabs = args[:ntab]
        srcr, dstr = args[ntab], args[ntab + 1]
        outs = args[ntab + 2:2 * ntab + 2]
        src_v, dst_v, bufs, zrow_v, acc = args[2 * ntab + 2:2 * ntab + 7]
        sems = args[2 * ntab + 7:]
        c = lax.axis_index("c")
        s = lax.axis_index("s")
        wid = c * 16 + s
        pltpu.sync_copy(srcr.at[wid], src_v)
        pltpu.sync_copy(dstr.at[wid], dst_v)

        def zfill(i, carry):
            for kk in range(W // 16):
                zrow_v[i, pl.ds(kk * 16, 16)] = jnp.zeros((16,), jnp.float32)
            return carry
        lax.fori_loop(0, ZC, zfill, 0)

        for ti in range(ntab):
            tab = tabs[ti]
            for b in range(nbuf):
                pltpu.async_copy(tab.at[src_v.at[b]], bufs.at[b], sems[b])

            for kk in range(RB // ZC):
                pltpu.sync_copy(zrow_v, acc.at[pl.ds(s * RB + kk * ZC, ZC)])

            @pl.when(s == 15)
            def _():
                pltpu.sync_copy(zrow_v.at[pl.ds(0, RT)],
                                acc.at[pl.ds(16 * RB, RT)])
            plsc.subcore_barrier()

            def body(g, carry):
                for b in range(nbuf):
                    j = g * nbuf + b
                    pltpu.make_async_copy(
                        tab.at[pl.ds(0, EB)], bufs.at[b], sems[b]).wait()
                    pltpu.sync_copy(bufs.at[b], acc.at[dst_v.at[j]], add=True)

                    @pl.when(g < NBLK // nbuf - 1)
                    def _(b=b, j=j):
                        pltpu.async_copy(tab.at[src_v.at[j + nbuf]],
                                         bufs.at[b], sems[b])
                return carry
            lax.fori_loop(0, NBLK // nbuf, body, 0)
            plsc.subcore_barrier()
            pltpu.sync_copy(acc.at[pl.ds(s * RB, RB)],
                            outs[ti].at[c, pl.ds(s * RB, RB)])

            @pl.when(s == 15)
            def _():
                pltpu.sync_copy(acc.at[pl.ds(16 * RB, RT)],
                                outs[ti].at[c, pl.ds(16 * RB, RT)])
            plsc.subcore_barrier()

    return k


_sc_passA = _make_sc_pass(3 * HID, 1, 2)
_sc_passB = _make_sc_pass(3 * HID, 3, 2)
_sc_pass1 = _make_sc_pass(HID, 1, 5)


def _dinv_of(degp_ref):
    deg = degp_ref[0, :, 0:1] + degp_ref[1, :, 0:1]
    return jnp.where(deg > 0, lax.rsqrt(deg), 0.0)


def _prep_body(x_ref, w0_ref, w1_ref, b_ref, degp_ref, xp0_ref, xpre_ref):
    xb = x_ref[0]
    dinv = _dinv_of(degp_ref)
    xp0_ref[0] = jnp.dot(xb, w0_ref[...], preferred_element_type=jnp.float32) + b_ref[...]
    xpre_ref[0] = dinv * jnp.dot(xb, w1_ref[...], preferred_element_type=jnp.float32)


def _prep(x, w0cat, w1cat, bcat, degp, t_off, nt):
    return pl.pallas_call(
        _prep_body,
        grid=(nt, NB),
        in_specs=[
            pl.BlockSpec((1, BLK, F_IN), lambda t, i, t_off=t_off: (t + t_off, i, 0)),
            pl.BlockSpec((F_IN, 3 * HID), lambda t, i: (0, 0)),
            pl.BlockSpec((F_IN, 3 * HID), lambda t, i: (0, 0)),
            pl.BlockSpec((1, 3 * HID), lambda t, i: (0, 0)),
            pl.BlockSpec((2, BLK, 16), lambda t, i: (0, i, 0)),
        ],
        out_specs=[
            pl.BlockSpec((1, BLK, 3 * HID), lambda t, i: (t, i, 0)),
            pl.BlockSpec((1, BLK, 3 * HID), lambda t, i: (t, i, 0)),
        ],
        out_shape=[
            jax.ShapeDtypeStruct((nt, N, 3 * HID), jnp.float32),
            jax.ShapeDtypeStruct((nt, N, 3 * HID), jnp.float32),
        ],
    )(x, w0cat, w1cat, bcat, degp)


def _t0_body(xp0_ref, sx_ref, degp_ref, h_ref, hp_ref):
    dinv = _dinv_of(degp_ref)
    pre = xp0_ref[0] - dinv * (sx_ref[0] + sx_ref[1])
    z = jax.nn.sigmoid(pre[:, 0:HID])
    ht = jnp.tanh(pre[:, 2 * HID:3 * HID])
    h = (1.0 - z) * ht
    h_ref[...] = h
    hp_ref[...] = dinv * h


def _t0(xp0, sx, degp):
    return pl.pallas_call(
        _t0_body,
        grid=(NB,),
        in_specs=[
            pl.BlockSpec((1, BLK, 3 * HID), lambda i: (0, i, 0)),
            pl.BlockSpec((2, BLK, 3 * HID), lambda i: (0, i, 0)),
            pl.BlockSpec((2, BLK, 16), lambda i: (0, i, 0)),
        ],
        out_specs=[
            pl.BlockSpec((BLK, HID), lambda i: (i, 0)),
            pl.BlockSpec((BLK, HID), lambda i: (i, 0)),
        ],
        out_shape=[
            jax.ShapeDtypeStruct((N, HID), jnp.float32),
            jax.ShapeDtypeStruct((N, HID), jnp.float32),
        ],
    )(xp0, sx, degp)


def _gates_body(xp0_ref, sx_ref, h_ref, sh_ref, degp_ref,
                wz0_ref, wr0_ref, wz1_ref, wr1_ref,
                z_ref, g_ref, gp_ref):
    dinv = _dinv_of(degp_ref)
    h = h_ref[...]
    lh = -dinv * (sh_ref[0] + sh_ref[1])
    xp = xp0_ref[0] - dinv * (sx_ref[0] + sx_ref[1])
    z = jax.nn.sigmoid(
        xp[:, 0:HID]
        + jnp.dot(h, wz0_ref[...], preferred_element_type=jnp.float32)
        + jnp.dot(lh, wz1_ref[...], preferred_element_type=jnp.float32))
    r = jax.nn.sigmoid(
        xp[:, HID:2 * HID]
        + jnp.dot(h, wr0_ref[...], preferred_element_type=jnp.float32)
        + jnp.dot(lh, wr1_ref[...], preferred_element_type=jnp.float32))
    g = h * r
    z_ref[...] = z
    g_ref[...] = g
    gp_ref[...] = dinv * g


def _gates(t, xp0, sx, h, sh, degp, wz0, wr0, wz1, wr1):
    return pl.pallas_call(
        _gates_body,
        grid=(NB,),
        in_specs=[
            pl.BlockSpec((1, BLK, 3 * HID), lambda i, t=t: (t, i, 0)),
            pl.BlockSpec((2, BLK, 3 * HID), lambda i: (0, i, 0)),
            pl.BlockSpec((BLK, HID), lambda i: (i, 0)),
            pl.BlockSpec((2, BLK, HID), lambda i: (0, i, 0)),
            pl.BlockSpec((2, BLK, 16), lambda i: (0, i, 0)),
            pl.BlockSpec((HID, HID), lambda i: (0, 0)),
            pl.BlockSpec((HID, HID), lambda i: (0, 0)),
            pl.BlockSpec((HID, HID), lambda i: (0, 0)),
            pl.BlockSpec((HID, HID), lambda i: (0, 0)),
        ],
        out_specs=[
            pl.BlockSpec((BLK, HID), lambda i: (i, 0)),
            pl.BlockSpec((BLK, HID), lambda i: (i, 0)),
            pl.BlockSpec((BLK, HID), lambda i: (i, 0)),
        ],
        out_shape=[
            jax.ShapeDtypeStruct((N, HID), jnp.float32),
            jax.ShapeDtypeStruct((N, HID), jnp.float32),
            jax.ShapeDtypeStruct((N, HID), jnp.float32),
        ],
    )(xp0, sx, h, sh, degp, wz0, wr0, wz1, wr1)


def _update_body(xp0_ref, sx_ref, z_ref, g_ref, sg_ref, h_ref, degp_ref,
                 wh0_ref, wh1_ref, hn_ref, hp_ref):
    dinv = _dinv_of(degp_ref)
    g = g_ref[...]
    lg = -dinv * (sg_ref[0] + sg_ref[1])
    xh = (xp0_ref[0, :, 2 * HID:3 * HID]
          - dinv * (sx_ref[0, :, 2 * HID:3 * HID] + sx_ref[1, :, 2 * HID:3 * HID]))
    ht = jnp.tanh(
        xh
        + jnp.dot(g, wh0_ref[...], preferred_element_type=jnp.float32)
        + jnp.dot(lg, wh1_ref[...], preferred_element_type=jnp.float32))
    z = z_ref[...]
    hn = z * h_ref[...] + (1.0 - z) * ht
    hn_ref[...] = hn
    hp_ref[...] = dinv * hn


def _update(t, xp0, sx, z, g, sg, h, degp, wh0, wh1):
    return pl.pallas_call(
        _update_body,
        grid=(NB,),
        in_specs=[
            pl.BlockSpec((1, BLK, 3 * HID), lambda i, t=t: (t, i, 0)),
            pl.BlockSpec((2, BLK, 3 * HID), lambda i: (0, i, 0)),
            pl.BlockSpec((BLK, HID), lambda i: (i, 0)),
            pl.BlockSpec((BLK, HID), lambda i: (i, 0)),
            pl.BlockSpec((2, BLK, HID), lambda i: (0, i, 0)),
            pl.BlockSpec((BLK, HID), lambda i: (i, 0)),
            pl.BlockSpec((2, BLK, 16), lambda i: (0, i, 0)),
            pl.BlockSpec((HID, HID), lambda i: (0, 0)),
            pl.BlockSpec((HID, HID), lambda i: (0, 0)),
        ],
        out_specs=[
            pl.BlockSpec((BLK, HID), lambda i: (i, 0)),
            pl.BlockSpec((BLK, HID), lambda i: (i, 0)),
        ],
        out_shape=[
            jax.ShapeDtypeStruct((N, HID), jnp.float32),
            jax.ShapeDtypeStruct((N, HID), jnp.float32),
        ],
    )(xp0, sx, z, g, sg, h, degp, wh0, wh1)


def _final_body(xp0_ref, sx_ref, z_ref, g_ref, sg_ref, h_ref, degp_ref,
                wh0_ref, wh1_ref, w1_ref, b1_ref, w2_ref, b2_ref,
                hn_ref, out_ref):
    dinv = _dinv_of(degp_ref)
    g = g_ref[...]
    lg = -dinv * (sg_ref[0] + sg_ref[1])
    xh = (xp0_ref[0, :, 2 * HID:3 * HID]
          - dinv * (sx_ref[0, :, 2 * HID:3 * HID] + sx_ref[1, :, 2 * HID:3 * HID]))
    ht = jnp.tanh(
        xh
        + jnp.dot(g, wh0_ref[...], preferred_element_type=jnp.float32)
        + jnp.dot(lg, wh1_ref[...], preferred_element_type=jnp.float32))
    z = z_ref[...]
    hn = z * h_ref[...] + (1.0 - z) * ht
    hn_ref[...] = hn
    h1 = jax.nn.relu(
        jnp.dot(hn, w1_ref[...], preferred_element_type=jnp.float32)
        + b1_ref[...])
    out_ref[...] = jnp.dot(h1, w2_ref[...], preferred_element_type=jnp.float32) + b2_ref[...]


def _final(t, xp0, sx, z, g, sg, h, degp, wh0, wh1, w1, b1, w2, b2):
    return pl.pallas_call(
        _final_body,
        grid=(NB,),
        in_specs=[
            pl.BlockSpec((1, BLK, 3 * HID), lambda i, t=t: (t, i, 0)),
            pl.BlockSpec((2, BLK, 3 * HID), lambda i: (0, i, 0)),
            pl.BlockSpec((BLK, HID), lambda i: (i, 0)),
            pl.BlockSpec((BLK, HID), lambda i: (i, 0)),
            pl.BlockSpec((2, BLK, HID), lambda i: (0, i, 0)),
            pl.BlockSpec((BLK, HID), lambda i: (i, 0)),
            pl.BlockSpec((2, BLK, 16), lambda i: (0, i, 0)),
            pl.BlockSpec((HID, HID), lambda i: (0, 0)),
            pl.BlockSpec((HID, HID), lambda i: (0, 0)),
            pl.BlockSpec((HID, HID), lambda i: (0, 0)),
            pl.BlockSpec((1, HID), lambda i: (0, 0)),
            pl.BlockSpec((HID, F_IN), lambda i: (0, 0)),
            pl.BlockSpec((1, F_IN), lambda i: (0, 0)),
        ],
        out_specs=[
            pl.BlockSpec((BLK, HID), lambda i: (i, 0)),
            pl.BlockSpec((BLK, F_IN), lambda i: (i, 0)),
        ],
        out_shape=[
            jax.ShapeDtypeStruct((N, HID), jnp.float32),
            jax.ShapeDtypeStruct((N, F_IN), jnp.float32),
        ],
    )(xp0, sx, z, g, sg, h, degp, wh0, wh1, w1, b1, w2, b2)


def kernel(x, edge_index, params):
    srcr = edge_index[0].reshape(NW, NBLK, EB)
    dstr = edge_index[1].reshape(NW, NBLK, EB)

    degp = _deg_kernel(srcr)

    w0cat = jnp.concatenate(
        [params['x_z']['W'][0], params['x_r']['W'][0], params['x_h']['W'][0]], axis=1)
    w1cat = jnp.concatenate(
        [params['x_z']['W'][1], params['x_r']['W'][1], params['x_h']['W'][1]], axis=1)
    bcat = jnp.concatenate(
        [params['x_z']['b'] + params['h_z']['b'],
         params['x_r']['b'] + params['h_r']['b'],
         params['x_h']['b'] + params['h_h']['b']]).reshape(1, 3 * HID)

    xp0a, xprea = _prep(x, w0cat, w1cat, bcat, degp, 0, 1)
    (sxp0,) = _sc_passA(xprea[0], srcr, dstr)
    xp0b, xpreb = _prep(x, w0cat, w1cat, bcat, degp, 1, 3)
    sxpB = _sc_passB(xpreb[0], xpreb[1], xpreb[2], srcr, dstr)

    h, hp = _t0(xp0a, sxp0, degp)

    for t in range(1, T):
        (shp,) = _sc_pass1(hp, srcr, dstr)
        z, g, gp = _gates(t - 1, xp0b, sxpB[t - 1], h, shp, degp,
                          params['h_z']['W'][0], params['h_r']['W'][0],
                          params['h_z']['W'][1], params['h_r']['W'][1])
        (sgp,) = _sc_pass1(gp, srcr, dstr)
        if t < T - 1:
            h, hp = _update(t - 1, xp0b, sxpB[t - 1], z, g, sgp, h, degp,
                            params['h_h']['W'][0], params['h_h']['W'][1])
        else:
            h, x_pred = _final(t - 1, xp0b, sxpB[t - 1], z, g, sgp, h, degp,
                               params['h_h']['W'][0], params['h_h']['W'][1],
                               params['dec_W1'], params['dec_b1'].reshape(1, HID),
                               params['dec_W2'], params['dec_b2'].reshape(1, F_IN))

    return (x_pred, h)
